# Initial kernel scaffold; baseline (speedup 1.0000x reference)
import jax
import jax.numpy as jnp
from jax.experimental import pallas as pl


def kernel(x, edge_index, edge_attr, batch, W1, a1_src, a1_dst, b1, W2, a2_src, a2_dst, b2, Wfc, bfc):
    raise NotImplementedError("write your pallas kernel here")


# speedup vs baseline: 35.0814x; 35.0814x over previous
import functools

import jax
import jax.numpy as jnp
from jax import lax
from jax.experimental import pallas as pl
from jax.experimental.pallas import tpu as pltpu
from jax.experimental.pallas import tpu_sc as plsc

_NC = 2
_NS = 16
_L = 16
_K = 128
_G = 256


def _cdiv(a, b):
    return (a + b - 1) // b


_IC = 112


@functools.lru_cache(maxsize=None)
def _sc_edge_pass(n_pad, e_pad):
    half = n_pad // _NC
    rows_pt = half // _NS
    init_chunks = rows_pt // _IC
    epc = e_pad // (_NS * _K)

    mesh = plsc.VectorSubcoreMesh(
        core_axis_name="c", subcore_axis_name="s",
        num_cores=_NC, num_subcores=_NS)

    @functools.partial(
        pl.kernel,
        out_type=jax.ShapeDtypeStruct((_NC, half, 16), jnp.float32),
        mesh=mesh,
        compiler_params=pltpu.CompilerParams(
            needs_layout_passes=False, use_tc_tiling_on_sc=False),
        scratch_types=[
            pltpu.VMEM((n_pad,), jnp.float32),
            pltpu.VMEM((n_pad,), jnp.float32),
            pltpu.VMEM((_K,), jnp.int32),
            pltpu.VMEM((_K,), jnp.int32),
            pltpu.VMEM((_K,), jnp.int32),
            pltpu.VMEM((_K, 16), jnp.float32),
            pltpu.VMEM((_K, 16), jnp.float32),
            pltpu.VMEM((_L,), jnp.float32),
            pltpu.VMEM_SHARED((half, 16), jnp.float32),
            pltpu.SemaphoreType.DMA,
        ],
    )
    def edge_pass(src_hbm, dst_hbm, as_hbm, ad_hbm, h16_hbm, mas_hbm,
                  out_hbm, as_v, ad_v, src_v, dst_v, ldst_v, rows_v, out_v,
                  mas_v, acc, sem):
        core = lax.axis_index("c")
        sub = lax.axis_index("s")

        pltpu.sync_copy(as_hbm, as_v)
        pltpu.sync_copy(ad_hbm, ad_v)
        pltpu.sync_copy(mas_hbm, mas_v)
        mas = mas_v[...]
        rbase = sub * rows_pt
        nbase = core * half + rbase

        def init_body(c, _):
            base = nbase + c * _IC
            pltpu.sync_copy(h16_hbm.at[pl.ds(base, _IC)],
                            rows_v.at[pl.ds(0, _IC)])
            for g in range(_IC // _L):
                nb = base + g * _L
                a_s = as_v[pl.ds(nb, _L)]
                a_d = ad_v[pl.ds(nb, _L)]
                z = a_s + a_d
                e = jnp.maximum(z, 0.2 * z)
                mh = jnp.maximum(0.0, mas + a_d)
                w16 = jnp.exp(e - mh)
                for j in range(_L):
                    out_v[g * _L + j, :] = rows_v[g * _L + j, :] * w16[j]
            pltpu.sync_copy(out_v.at[pl.ds(0, _IC)],
                            acc.at[pl.ds(rbase + c * _IC, _IC)])
            return 0

        lax.fori_loop(0, init_chunks, init_body, 0)

        plsc.subcore_barrier()

        ebase0 = sub * (epc * _K)
        lo = core * half

        def edge_body(c, _):
            eb = ebase0 + c * _K
            pltpu.sync_copy(src_hbm.at[pl.ds(eb, _K)], src_v)
            pltpu.sync_copy(dst_hbm.at[pl.ds(eb, _K)], dst_v)
            cp = pltpu.async_copy(h16_hbm.at[src_v], rows_v, sem)
            ws = []
            for g in range(_K // _L):
                s16 = src_v[pl.ds(g * _L, _L)]
                d16 = dst_v[pl.ds(g * _L, _L)]
                a_s = plsc.load_gather(as_v, [s16])
                a_d = plsc.load_gather(ad_v, [d16])
                z = a_s + a_d
                e = jnp.maximum(z, 0.2 * z)
                mh = jnp.maximum(0.0, mas + a_d)
                w16 = jnp.exp(e - mh)
                ld = d16 - lo
                keep = (ld >= 0) & (ld < half)
                w16 = jnp.where(keep, w16, 0.0)
                ldst_v[pl.ds(g * _L, _L)] = jnp.clip(ld, 0, half - 1)
                ws.append(w16)
            cp.wait()
            for g in range(_K // _L):
                w16 = ws[g]
                for j in range(_L):
                    out_v[g * _L + j, :] = rows_v[g * _L + j, :] * w16[j]
            pltpu.sync_copy(out_v, acc.at[ldst_v], add=True)
            return 0

        lax.fori_loop(0, epc, edge_body, 0)

        plsc.subcore_barrier()

        pltpu.sync_copy(acc.at[pl.ds(rbase, rows_pt)],
                        out_hbm.at[core, pl.ds(rbase, rows_pt)])

    return edge_pass


def _prep1(x_pad, W1, a1s, a1d, n, n_pad, blk):
    nb = n_pad // blk
    d_in = x_pad.shape[1]

    def body(x_ref, w_ref, s_ref, d_ref, h16_ref, as_ref, ad_ref, bm_ref):
        i = pl.program_id(0)
        h = lax.dot_general(x_ref[...], w_ref[...],
                            (((1,), (0,)), ((), ())),
                            preferred_element_type=jnp.float32)
        asv = lax.dot_general(h, s_ref[...], (((1,), (0,)), ((), ())))
        adv = lax.dot_general(h, d_ref[...], (((1,), (0,)), ((), ())))
        ridx = lax.broadcasted_iota(jnp.int32, (blk, 1), 0) + i * blk
        valid = (ridx < n).astype(jnp.float32)
        h16_ref[...] = jnp.concatenate(
            [h, valid, jnp.zeros((blk, 7), jnp.float32)], axis=1)
        as_ref[...] = asv
        ad_ref[...] = adv
        bm_ref[...] = jnp.max(asv, axis=0, keepdims=True)[None]

    return pl.pallas_call(
        body,
        grid=(nb,),
        in_specs=[
            pl.BlockSpec((blk, d_in), lambda i: (i, 0)),
            pl.BlockSpec((d_in, 8), lambda i: (0, 0)),
            pl.BlockSpec((8, 1), lambda i: (0, 0)),
            pl.BlockSpec((8, 1), lambda i: (0, 0)),
        ],
        out_specs=[
            pl.BlockSpec((blk, 16), lambda i: (i, 0)),
            pl.BlockSpec((blk, 1), lambda i: (i, 0)),
            pl.BlockSpec((blk, 1), lambda i: (i, 0)),
            pl.BlockSpec((1, 1, 1), lambda i: (i, 0, 0)),
        ],
        out_shape=[
            jax.ShapeDtypeStruct((n_pad, 16), jnp.float32),
            jax.ShapeDtypeStruct((n_pad, 1), jnp.float32),
            jax.ShapeDtypeStruct((n_pad, 1), jnp.float32),
            jax.ShapeDtypeStruct((nb, 1, 1), jnp.float32),
        ],
    )(x_pad, W1, a1s, a1d)


def _mid(p1, b1, W2, a2s, a2d, n, n_pad, blk):
    nb = n_pad // blk

    def body(p_ref, b1_ref, w2_ref, s_ref, d_ref,
             h16_ref, as_ref, ad_ref, bm_ref):
        i = pl.program_id(0)
        tot = p_ref[...]
        s = tot[:, 8:9]
        h1r = jnp.maximum(tot[:, :8] / (s + 1e-16) + b1_ref[...], 0.0)
        ridx = lax.broadcasted_iota(jnp.int32, (blk, 1), 0) + i * blk
        validf = (ridx < n).astype(jnp.float32)
        h1r = h1r * validf
        ats = lax.dot_general(w2_ref[...], s_ref[...],
                              (((1,), (0,)), ((), ())))
        atd = lax.dot_general(w2_ref[...], d_ref[...],
                              (((1,), (0,)), ((), ())))
        asv = lax.dot_general(h1r, ats, (((1,), (0,)), ((), ())))
        adv = lax.dot_general(h1r, atd, (((1,), (0,)), ((), ())))
        h16_ref[...] = jnp.concatenate(
            [h1r, validf, jnp.zeros((blk, 7), jnp.float32)], axis=1)
        as_ref[...] = asv
        ad_ref[...] = adv
        bm_ref[...] = jnp.max(asv, axis=0, keepdims=True)[None]

    return pl.pallas_call(
        body,
        grid=(nb,),
        in_specs=[
            pl.BlockSpec((blk, 16), lambda i: (i, 0)),
            pl.BlockSpec((1, 8), lambda i: (0, 0)),
            pl.BlockSpec((8, _G), lambda i: (0, 0)),
            pl.BlockSpec((_G, 1), lambda i: (0, 0)),
            pl.BlockSpec((_G, 1), lambda i: (0, 0)),
        ],
        out_specs=[
            pl.BlockSpec((blk, 16), lambda i: (i, 0)),
            pl.BlockSpec((blk, 1), lambda i: (i, 0)),
            pl.BlockSpec((blk, 1), lambda i: (i, 0)),
            pl.BlockSpec((1, 1, 1), lambda i: (i, 0, 0)),
        ],
        out_shape=[
            jax.ShapeDtypeStruct((n_pad, 16), jnp.float32),
            jax.ShapeDtypeStruct((n_pad, 1), jnp.float32),
            jax.ShapeDtypeStruct((n_pad, 1), jnp.float32),
            jax.ShapeDtypeStruct((nb, 1, 1), jnp.float32),
        ],
    )(p1, b1, W2, a2s, a2d)


def _final(p2, W2, b2, Wfc, bfc, batch_pad, n, n_pad, blk):
    nb = n_pad // blk

    def body(p_ref, w2_ref, b2_ref, wfc_ref, bfc_ref, b_ref, out_ref):
        i = pl.program_id(0)
        tot = p_ref[...]
        s = tot[:, 8:9]
        agg = tot[:, :8] / (s + 1e-16)
        h2 = jnp.maximum(
            lax.dot_general(agg, w2_ref[...], (((1,), (0,)), ((), ())),
                            preferred_element_type=jnp.float32)
            + b2_ref[...], 0.0)
        y = lax.dot_general(h2, wfc_ref[...], (((1,), (0,)), ((), ())))
        y = y + bfc_ref[...]
        ridx = lax.broadcasted_iota(jnp.int32, (blk, 1), 0) + i * blk
        y = jnp.where(ridx < n, y, 0.0)
        oh = (b_ref[...] == lax.broadcasted_iota(
            jnp.int32, (1, _G), 1)).astype(jnp.float32)
        contrib = lax.dot_general(oh, y, (((0,), (0,)), ((), ())))

        @pl.when(i == 0)
        def _():
            out_ref[...] = contrib

        @pl.when(i != 0)
        def _():
            out_ref[...] = out_ref[...] + contrib

    return pl.pallas_call(
        body,
        grid=(nb,),
        in_specs=[
            pl.BlockSpec((blk, 16), lambda i: (i, 0)),
            pl.BlockSpec((8, _G), lambda i: (0, 0)),
            pl.BlockSpec((1, _G), lambda i: (0, 0)),
            pl.BlockSpec((_G, 1), lambda i: (0, 0)),
            pl.BlockSpec((1, 1), lambda i: (0, 0)),
            pl.BlockSpec((blk, 1), lambda i: (i, 0)),
        ],
        out_specs=pl.BlockSpec((_G, 1), lambda i: (0, 0)),
        out_shape=jax.ShapeDtypeStruct((_G, 1), jnp.float32),
    )(p2, W2, b2, Wfc, bfc, batch_pad)


def kernel(x, edge_index, edge_attr, batch, W1, a1_src, a1_dst, b1,
           W2, a2_src, a2_dst, b2, Wfc, bfc):
    n = x.shape[0]
    e = edge_index.shape[1]
    rows_pt = _cdiv(n, _NC * _NS * _IC) * _IC
    n_pad = rows_pt * _NS * _NC
    epc = _cdiv(e, _NS * _K)
    e_pad = epc * _NS * _K
    blk = n_pad // 8

    src = jnp.concatenate(
        [edge_index[0], jnp.zeros((e_pad - e,), jnp.int32)])
    dst = jnp.concatenate(
        [edge_index[1], jnp.full((e_pad - e,), n, jnp.int32)])
    x_pad = jnp.zeros((n_pad, x.shape[1]), jnp.float32).at[:n].set(x)
    batch_pad = jnp.zeros((n_pad, 1), jnp.int32).at[:n, 0].set(batch)

    h16_1, as1, ad1, bm1 = _prep1(
        x_pad, W1, a1_src.reshape(8, 1), a1_dst.reshape(8, 1), n, n_pad, blk)
    mas1 = jnp.broadcast_to(jnp.max(bm1), (_L,))

    sc = _sc_edge_pass(n_pad, e_pad)
    p1 = sc(src, dst, as1[:, 0], ad1[:, 0], h16_1, mas1)
    p1 = p1.reshape(n_pad, 16)

    h16_2, as2, ad2, bm2 = _mid(
        p1, b1.reshape(1, 8), W2, a2_src.reshape(_G, 1),
        a2_dst.reshape(_G, 1), n, n_pad, blk)
    mas2 = jnp.broadcast_to(jnp.max(bm2), (_L,))

    p2 = sc(src, dst, as2[:, 0], ad2[:, 0], h16_2, mas2)
    p2 = p2.reshape(n_pad, 16)

    return _final(p2, W2, b2.reshape(1, _G), Wfc, bfc.reshape(1, 1),
                  batch_pad, n, n_pad, blk)


# speedup vs baseline: 49.7685x; 1.4187x over previous
import functools

import jax
import jax.numpy as jnp
from jax import lax
from jax.experimental import pallas as pl
from jax.experimental.pallas import tpu as pltpu
from jax.experimental.pallas import tpu_sc as plsc

_NC = 2
_NS = 16
_L = 16
_K = 128
_G = 256


def _cdiv(a, b):
    return (a + b - 1) // b


_IC = 112


@functools.lru_cache(maxsize=None)
def _sc_edge_pass(n_pad, e_pad):
    half = n_pad // _NC
    rows_pt = half // _NS
    init_chunks = rows_pt // _IC
    epc = e_pad // (_NS * _K)

    mesh = plsc.VectorSubcoreMesh(
        core_axis_name="c", subcore_axis_name="s",
        num_cores=_NC, num_subcores=_NS)

    @functools.partial(
        pl.kernel,
        out_type=jax.ShapeDtypeStruct((_NC, half, 16), jnp.float32),
        mesh=mesh,
        compiler_params=pltpu.CompilerParams(
            needs_layout_passes=False, use_tc_tiling_on_sc=False),
        scratch_types=[
            pltpu.VMEM((n_pad,), jnp.float32),
            pltpu.VMEM((n_pad // _NC,), jnp.float32),
            pltpu.VMEM((2, _K), jnp.int32),
            pltpu.VMEM((2, _K), jnp.int32),
            pltpu.VMEM((_K,), jnp.int32),
            pltpu.VMEM((2, _K, 16), jnp.float32),
            pltpu.VMEM((_K, 16), jnp.float32),
            pltpu.VMEM((_L,), jnp.float32),
            pltpu.VMEM_SHARED((half, 16), jnp.float32),
            pltpu.SemaphoreType.DMA,
            pltpu.SemaphoreType.DMA,
            pltpu.SemaphoreType.DMA,
            pltpu.SemaphoreType.DMA,
        ],
    )
    def edge_pass(src_hbm, dst_hbm, as_hbm, ad_hbm, h16_hbm, mas_hbm,
                  out_hbm, as_v, ad_v, src_v, dst_v, ldst_v, rows_v, out_v,
                  mas_v, acc, isem0, isem1, gsem0, gsem1):
        core = lax.axis_index("c")
        sub = lax.axis_index("s")
        lo = core * half

        pltpu.sync_copy(as_hbm, as_v)
        pltpu.sync_copy(ad_hbm.at[pl.ds(lo, half)], ad_v)
        pltpu.sync_copy(mas_hbm, mas_v)
        mas = mas_v[...]
        rbase = sub * rows_pt
        nbase = lo + rbase

        def init_body(c, _):
            base = nbase + c * _IC
            lbase = rbase + c * _IC
            pltpu.sync_copy(h16_hbm.at[pl.ds(base, _IC)],
                            rows_v.at[0, pl.ds(0, _IC)])
            for g in range(_IC // _L):
                a_s = as_v[pl.ds(base + g * _L, _L)]
                a_d = ad_v[pl.ds(lbase + g * _L, _L)]
                z = a_s + a_d
                e = jnp.maximum(z, 0.2 * z)
                mh = jnp.maximum(0.0, mas + a_d)
                w16 = jnp.exp(e - mh)
                for j in range(_L):
                    out_v[g * _L + j, :] = rows_v[0, g * _L + j, :] * w16[j]
            pltpu.sync_copy(out_v.at[pl.ds(0, _IC)],
                            acc.at[pl.ds(lbase, _IC)])
            return 0

        lax.fori_loop(0, init_chunks, init_body, 0)

        plsc.subcore_barrier()

        ebase0 = sub * (epc * _K)
        isems = (isem0, isem1)
        gsems = (gsem0, gsem1)

        def issue_idx(c, b):
            eb = ebase0 + c * _K
            pltpu.async_copy(src_hbm.at[pl.ds(eb, _K)], src_v.at[b], isems[b])
            pltpu.async_copy(dst_hbm.at[pl.ds(eb, _K)], dst_v.at[b], isems[b])

        def wait_idx(c, b):
            eb = ebase0 + c * _K
            pltpu.make_async_copy(
                src_hbm.at[pl.ds(eb, _K)], src_v.at[b], isems[b]).wait()
            pltpu.make_async_copy(
                dst_hbm.at[pl.ds(eb, _K)], dst_v.at[b], isems[b]).wait()

        def issue_gather(b):
            pltpu.async_copy(h16_hbm.at[src_v.at[b]], rows_v.at[b], gsems[b])

        def wait_gather(b):
            pltpu.make_async_copy(
                h16_hbm.at[src_v.at[b]], rows_v.at[b], gsems[b]).wait()

        issue_idx(0, 0)
        wait_idx(0, 0)
        issue_gather(0)
        issue_idx(1, 1)

        last_t = epc // 2 - 1

        def edge_pair(t, _):
            for b in range(2):
                c = 2 * t + b
                ws = []
                for g in range(_K // _L):
                    s16 = src_v[b, pl.ds(g * _L, _L)]
                    d16 = dst_v[b, pl.ds(g * _L, _L)]
                    ld = d16 - lo
                    ldc = jnp.clip(ld, 0, half - 1)
                    a_s = plsc.load_gather(as_v, [s16])
                    a_d = plsc.load_gather(ad_v, [ldc])
                    z = a_s + a_d
                    e = jnp.maximum(z, 0.2 * z)
                    mh = jnp.maximum(0.0, mas + a_d)
                    w16 = jnp.exp(e - mh)
                    keep = (ld >= 0) & (ld < half)
                    w16 = jnp.where(keep, w16, 0.0)
                    ldst_v[pl.ds(g * _L, _L)] = ldc
                    ws.append(w16)
                if b == 0:
                    wait_idx(c + 1, 1)
                    issue_gather(1)
                else:
                    @pl.when(t < last_t)
                    def _():
                        wait_idx(c + 1, 0)
                        issue_gather(0)
                wait_gather(b)
                if b == 0:
                    @pl.when(t < last_t)
                    def _():
                        issue_idx(c + 2, 0)
                else:
                    @pl.when(t < last_t)
                    def _():
                        issue_idx(c + 2, 1)
                for g in range(_K // _L):
                    w16 = ws[g]
                    for j in range(_L):
                        out_v[g * _L + j, :] = rows_v[b, g * _L + j, :] * w16[j]
                pltpu.sync_copy(out_v, acc.at[ldst_v], add=True)
            return 0

        lax.fori_loop(0, epc // 2, edge_pair, 0)

        plsc.subcore_barrier()

        pltpu.sync_copy(acc.at[pl.ds(rbase, rows_pt)],
                        out_hbm.at[core, pl.ds(rbase, rows_pt)])

    return edge_pass


def _prep1(x_pad, W1, a1s, a1d, n, n_pad, blk):
    nb = n_pad // blk
    d_in = x_pad.shape[1]

    def body(x_ref, w_ref, s_ref, d_ref, h16_ref, as_ref, ad_ref, bm_ref):
        i = pl.program_id(0)
        h = lax.dot_general(x_ref[...], w_ref[...],
                            (((1,), (0,)), ((), ())),
                            preferred_element_type=jnp.float32)
        asv = lax.dot_general(h, s_ref[...], (((1,), (0,)), ((), ())))
        adv = lax.dot_general(h, d_ref[...], (((1,), (0,)), ((), ())))
        ridx = lax.broadcasted_iota(jnp.int32, (blk, 1), 0) + i * blk
        valid = (ridx < n).astype(jnp.float32)
        h16_ref[...] = jnp.concatenate(
            [h, valid, jnp.zeros((blk, 7), jnp.float32)], axis=1)
        as_ref[...] = asv
        ad_ref[...] = adv
        bm_ref[...] = jnp.max(asv, axis=0, keepdims=True)[None]

    return pl.pallas_call(
        body,
        grid=(nb,),
        in_specs=[
            pl.BlockSpec((blk, d_in), lambda i: (i, 0)),
            pl.BlockSpec((d_in, 8), lambda i: (0, 0)),
            pl.BlockSpec((8, 1), lambda i: (0, 0)),
            pl.BlockSpec((8, 1), lambda i: (0, 0)),
        ],
        out_specs=[
            pl.BlockSpec((blk, 16), lambda i: (i, 0)),
            pl.BlockSpec((blk, 1), lambda i: (i, 0)),
            pl.BlockSpec((blk, 1), lambda i: (i, 0)),
            pl.BlockSpec((1, 1, 1), lambda i: (i, 0, 0)),
        ],
        out_shape=[
            jax.ShapeDtypeStruct((n_pad, 16), jnp.float32),
            jax.ShapeDtypeStruct((n_pad, 1), jnp.float32),
            jax.ShapeDtypeStruct((n_pad, 1), jnp.float32),
            jax.ShapeDtypeStruct((nb, 1, 1), jnp.float32),
        ],
    )(x_pad, W1, a1s, a1d)


def _mid(p1, b1, W2, a2s, a2d, n, n_pad, blk):
    nb = n_pad // blk

    def body(p_ref, b1_ref, w2_ref, s_ref, d_ref,
             h16_ref, as_ref, ad_ref, bm_ref):
        i = pl.program_id(0)
        tot = p_ref[...]
        s = tot[:, 8:9]
        h1r = jnp.maximum(tot[:, :8] / (s + 1e-16) + b1_ref[...], 0.0)
        ridx = lax.broadcasted_iota(jnp.int32, (blk, 1), 0) + i * blk
        validf = (ridx < n).astype(jnp.float32)
        h1r = h1r * validf
        ats = lax.dot_general(w2_ref[...], s_ref[...],
                              (((1,), (0,)), ((), ())))
        atd = lax.dot_general(w2_ref[...], d_ref[...],
                              (((1,), (0,)), ((), ())))
        asv = lax.dot_general(h1r, ats, (((1,), (0,)), ((), ())))
        adv = lax.dot_general(h1r, atd, (((1,), (0,)), ((), ())))
        h16_ref[...] = jnp.concatenate(
            [h1r, validf, jnp.zeros((blk, 7), jnp.float32)], axis=1)
        as_ref[...] = asv
        ad_ref[...] = adv
        bm_ref[...] = jnp.max(asv, axis=0, keepdims=True)[None]

    return pl.pallas_call(
        body,
        grid=(nb,),
        in_specs=[
            pl.BlockSpec((blk, 16), lambda i: (i, 0)),
            pl.BlockSpec((1, 8), lambda i: (0, 0)),
            pl.BlockSpec((8, _G), lambda i: (0, 0)),
            pl.BlockSpec((_G, 1), lambda i: (0, 0)),
            pl.BlockSpec((_G, 1), lambda i: (0, 0)),
        ],
        out_specs=[
            pl.BlockSpec((blk, 16), lambda i: (i, 0)),
            pl.BlockSpec((blk, 1), lambda i: (i, 0)),
            pl.BlockSpec((blk, 1), lambda i: (i, 0)),
            pl.BlockSpec((1, 1, 1), lambda i: (i, 0, 0)),
        ],
        out_shape=[
            jax.ShapeDtypeStruct((n_pad, 16), jnp.float32),
            jax.ShapeDtypeStruct((n_pad, 1), jnp.float32),
            jax.ShapeDtypeStruct((n_pad, 1), jnp.float32),
            jax.ShapeDtypeStruct((nb, 1, 1), jnp.float32),
        ],
    )(p1, b1, W2, a2s, a2d)


def _final(p2, W2, b2, Wfc, bfc, batch_pad, n, n_pad, blk):
    nb = n_pad // blk

    def body(p_ref, w2_ref, b2_ref, wfc_ref, bfc_ref, b_ref, out_ref):
        i = pl.program_id(0)
        tot = p_ref[...]
        s = tot[:, 8:9]
        agg = tot[:, :8] / (s + 1e-16)
        h2 = jnp.maximum(
            lax.dot_general(agg, w2_ref[...], (((1,), (0,)), ((), ())),
                            preferred_element_type=jnp.float32)
            + b2_ref[...], 0.0)
        y = lax.dot_general(h2, wfc_ref[...], (((1,), (0,)), ((), ())))
        y = y + bfc_ref[...]
        ridx = lax.broadcasted_iota(jnp.int32, (blk, 1), 0) + i * blk
        y = jnp.where(ridx < n, y, 0.0)
        oh = (b_ref[...] == lax.broadcasted_iota(
            jnp.int32, (1, _G), 1)).astype(jnp.float32)
        contrib = lax.dot_general(oh, y, (((0,), (0,)), ((), ())))

        @pl.when(i == 0)
        def _():
            out_ref[...] = contrib

        @pl.when(i != 0)
        def _():
            out_ref[...] = out_ref[...] + contrib

    return pl.pallas_call(
        body,
        grid=(nb,),
        in_specs=[
            pl.BlockSpec((blk, 16), lambda i: (i, 0)),
            pl.BlockSpec((8, _G), lambda i: (0, 0)),
            pl.BlockSpec((1, _G), lambda i: (0, 0)),
            pl.BlockSpec((_G, 1), lambda i: (0, 0)),
            pl.BlockSpec((1, 1), lambda i: (0, 0)),
            pl.BlockSpec((blk, 1), lambda i: (i, 0)),
        ],
        out_specs=pl.BlockSpec((_G, 1), lambda i: (0, 0)),
        out_shape=jax.ShapeDtypeStruct((_G, 1), jnp.float32),
    )(p2, W2, b2, Wfc, bfc, batch_pad)


def kernel(x, edge_index, edge_attr, batch, W1, a1_src, a1_dst, b1,
           W2, a2_src, a2_dst, b2, Wfc, bfc):
    n = x.shape[0]
    e = edge_index.shape[1]
    rows_pt = _cdiv(n, _NC * _NS * _IC) * _IC
    n_pad = rows_pt * _NS * _NC
    epc = _cdiv(e, _NS * _K * 2) * 2
    e_pad = epc * _NS * _K
    blk = n_pad // 8

    src = jnp.concatenate(
        [edge_index[0], jnp.zeros((e_pad - e,), jnp.int32)])
    dst = jnp.concatenate(
        [edge_index[1], jnp.full((e_pad - e,), n, jnp.int32)])
    x_pad = jnp.zeros((n_pad, x.shape[1]), jnp.float32).at[:n].set(x)
    batch_pad = jnp.zeros((n_pad, 1), jnp.int32).at[:n, 0].set(batch)

    h16_1, as1, ad1, bm1 = _prep1(
        x_pad, W1, a1_src.reshape(8, 1), a1_dst.reshape(8, 1), n, n_pad, blk)
    mas1 = jnp.broadcast_to(jnp.max(bm1), (_L,))

    sc = _sc_edge_pass(n_pad, e_pad)
    p1 = sc(src, dst, as1[:, 0], ad1[:, 0], h16_1, mas1)
    p1 = p1.reshape(n_pad, 16)

    h16_2, as2, ad2, bm2 = _mid(
        p1, b1.reshape(1, 8), W2, a2_src.reshape(_G, 1),
        a2_dst.reshape(_G, 1), n, n_pad, blk)
    mas2 = jnp.broadcast_to(jnp.max(bm2), (_L,))

    p2 = sc(src, dst, as2[:, 0], ad2[:, 0], h16_2, mas2)
    p2 = p2.reshape(n_pad, 16)

    return _final(p2, W2, b2.reshape(1, _G), Wfc, bfc.reshape(1, 1),
                  batch_pad, n, n_pad, blk)
